# baseline (speedup 1.0000x reference)
import jax
import jax.numpy as jnp
from jax.experimental import pallas as pl

K = 16


def _copy_kernel(x_ref, o_ref):
    o_ref[...] = x_ref[...]


def kernel(query_coords, query_features, key_coords, key_features):
    B, C, Nq = query_features.shape
    Nk = key_features.shape[2]
    q = jnp.transpose(query_coords, (0, 2, 1))
    kk = jnp.transpose(key_coords, (0, 2, 1))
    dist = (jnp.sum(q * q, axis=-1, keepdims=True)
            - 2.0 * jnp.einsum('bqc,bkc->bqk', q, kk)
            + jnp.sum(kk * kk, axis=-1)[:, None, :])
    _, idx = jax.lax.top_k(-dist, K)
    batch_offsets = jnp.arange(B).reshape(-1, 1, 1) * Nk
    flat_idx = (idx + batch_offsets).reshape(-1)
    key_flat = jnp.transpose(key_features, (0, 2, 1)).reshape(B * Nk, C)
    nf = key_flat[flat_idx, :]
    nf = jnp.transpose(nf.reshape(B, K, Nq, C), (0, 3, 2, 1))
    qf = jnp.broadcast_to(query_features[:, :, :, None], (B, C, Nq, K))
    edge = jnp.concatenate([nf - qf, qf], axis=1)
    edge2 = edge.reshape(B * 2 * C, Nq * K)
    out = pl.pallas_call(
        _copy_kernel,
        out_shape=jax.ShapeDtypeStruct(edge2.shape, edge2.dtype),
        grid=(64,),
        in_specs=[pl.BlockSpec((B * 2 * C // 64, Nq * K), lambda i: (i, 0))],
        out_specs=pl.BlockSpec((B * 2 * C // 64, Nq * K), lambda i: (i, 0)),
    )(edge2)
    return out.reshape(B, 2 * C, Nq, K)


# speedup vs baseline: 2.5312x; 2.5312x over previous
import functools

import jax
import jax.numpy as jnp
from jax import lax
from jax.experimental import pallas as pl
from jax.experimental.pallas import tpu as pltpu
from jax.experimental.pallas import tpu_sc as plsc

KNN = 16
LANES = 16
NC, NS = 2, 16
NW = NC * NS
NQ_BLK = 8

_GDN = lax.GatherDimensionNumbers(
    offset_dims=(), collapsed_slice_dims=(0,), start_index_map=(0,))


def _permute(v, idx):
    return lax.gather(v, idx[:, None], _GDN, (1,),
                      mode=lax.GatherScatterMode.PROMISE_IN_BOUNDS)


def _splat(v, lane):
    return _permute(v, jnp.full((LANES,), lane, jnp.int32))


def _tr_body(x_ref, o_ref):
    o_ref[...] = x_ref[0].T


def _transpose_key(key_features):
    B, C, Nk = key_features.shape
    TK = 512
    return pl.pallas_call(
        _tr_body,
        grid=(B, Nk // TK),
        in_specs=[pl.BlockSpec((1, C, TK), lambda b, t: (b, 0, t))],
        out_specs=pl.BlockSpec((TK, C), lambda b, t: (b * (Nk // TK) + t, 0)),
        out_shape=jax.ShapeDtypeStruct((B * Nk, C), jnp.float32),
    )(key_features)


def _tr_q_body(x_ref, o_ref):
    o_ref[0] = x_ref[0].T


def _transpose_qf(query_features):
    B, C, Nq = query_features.shape
    TQ = 512
    return pl.pallas_call(
        _tr_q_body,
        grid=(B, Nq // TQ),
        in_specs=[pl.BlockSpec((1, C, TQ), lambda b, t: (b, 0, t))],
        out_specs=pl.BlockSpec((1, TQ, C), lambda b, t: (b, t, 0)),
        out_shape=jax.ShapeDtypeStruct((B, Nq, C), jnp.float32),
    )(query_features)


def _knn_body(Nq, Nk, qc_hbm, kc_hbm, qcb_hbm, kcb_hbm, idx_hbm,
              kc_v, qc_v, kcb_v, qcb_v, kn2_v, oidx_v):
    nqw = Nq // (NW // 4)
    nchunk = Nk // LANES
    cid = lax.axis_index("c")
    sid = lax.axis_index("s")
    wid = cid * NS + sid
    b = wid // 8
    q0 = (wid % 8) * nqw
    pltpu.sync_copy(kc_hbm.at[b], kc_v)
    pltpu.sync_copy(qc_hbm.at[b, :, pl.ds(q0, nqw)], qc_v)
    pltpu.sync_copy(kcb_hbm.at[b], kcb_v)
    pltpu.sync_copy(qcb_hbm.at[b, :, pl.ds(q0, nqw)], qcb_v)
    iota = lax.iota(jnp.int32, LANES)
    lane15 = jnp.full((LANES,), 15, jnp.int32)
    inf = jnp.full((LANES,), jnp.inf, jnp.float32)
    zeros_i = jnp.zeros((LANES,), jnp.int32)

    def kn2_step(j, c):
        kx = kc_v[0, pl.ds(j * LANES, LANES)]
        ky = kc_v[1, pl.ds(j * LANES, LANES)]
        kz = kc_v[2, pl.ds(j * LANES, LANES)]
        kn2_v[pl.ds(j * LANES, LANES)] = kx * kx + ky * ky + kz * kz
        return c
    lax.fori_loop(0, nchunk, kn2_step, 0)

    def qgroup(qg, c0):
        qxv = qc_v[0, pl.ds(qg * LANES, LANES)]
        qyv = qc_v[1, pl.ds(qg * LANES, LANES)]
        qzv = qc_v[2, pl.ds(qg * LANES, LANES)]
        qxbv = qcb_v[0, pl.ds(qg * LANES, LANES)]
        qybv = qcb_v[1, pl.ds(qg * LANES, LANES)]
        qzbv = qcb_v[2, pl.ds(qg * LANES, LANES)]

        def one_query(l, c1):
            qxb = _splat(qxbv, l)
            qyb = _splat(qybv, l)
            qzb = _splat(qzbv, l)
            qx = _splat(qxv, l)
            qy = _splat(qyv, l)
            qz = _splat(qzv, l)
            qn2 = (qx * qx + qy * qy) + qz * qz

            def chunk_step(j, carry):
                pool_d, pool_i, thr = carry
                kx = kcb_v[0, pl.ds(j * LANES, LANES)]
                ky = kcb_v[1, pl.ds(j * LANES, LANES)]
                kz = kcb_v[2, pl.ds(j * LANES, LANES)]
                kn2 = kn2_v[pl.ds(j * LANES, LANES)]
                qk = (kx * qxb + ky * qyb) + kz * qzb
                d = (qn2 - 2.0 * qk) + kn2
                hit = jnp.any(d < thr)

                def merge(_):
                    ivec = iota + j * LANES
                    sd, si = plsc.sort_key_val(d, ivec)
                    rsd = lax.rev(sd, (0,))
                    rsi = lax.rev(si, (0,))
                    cm = rsd < pool_d
                    nd = jnp.where(cm, rsd, pool_d)
                    ni = jnp.where(cm, rsi, pool_i)
                    nd2, ni2 = plsc.sort_key_val(nd, ni)
                    return nd2, ni2, _permute(nd2, lane15)

                return lax.cond(hit, merge, lambda _: carry, 0)

            pool_d, pool_i, thr = lax.fori_loop(
                0, nchunk, chunk_step, (inf, zeros_i, inf))
            ql = qg * LANES + l
            row = (ql * 2) % nqw + iota // NQ_BLK
            colv = (ql // (nqw // 2)) * NQ_BLK + iota % NQ_BLK
            plsc.store_scatter(oidx_v, [row, colv], pool_i + b * Nk)
            return c1
        lax.fori_loop(0, LANES, one_query, c0)
        return c0
    lax.fori_loop(0, nqw // LANES, qgroup, 0)
    w = wid % 8
    pltpu.sync_copy(oidx_v, idx_hbm.at[b, :, pl.ds(w * LANES, LANES)])


def _knn_sc(query_coords, key_coords):
    B, _, Nq = query_coords.shape
    Nk = key_coords.shape[2]
    nqw = Nq // (NW // B)
    qcb = query_coords.astype(jnp.bfloat16).astype(jnp.float32)
    kcb = key_coords.astype(jnp.bfloat16).astype(jnp.float32)
    mesh = plsc.VectorSubcoreMesh(core_axis_name="c", subcore_axis_name="s")
    return pl.kernel(
        functools.partial(_knn_body, Nq, Nk),
        mesh=mesh,
        compiler_params=pltpu.CompilerParams(use_tc_tiling_on_sc=False, needs_layout_passes=False),
        out_type=jax.ShapeDtypeStruct((B, nqw, Nq * KNN // nqw), jnp.int32),
        scratch_types=[
            pltpu.VMEM((3, Nk), jnp.float32),
            pltpu.VMEM((3, nqw), jnp.float32),
            pltpu.VMEM((3, Nk), jnp.float32),
            pltpu.VMEM((3, nqw), jnp.float32),
            pltpu.VMEM((Nk,), jnp.float32),
            pltpu.VMEM((nqw, LANES), jnp.int32),
        ],
    )(query_coords, key_coords, qcb, kcb)


def _edge_body(B, C, Nq, kf_hbm, qf_hbm, idx_hbm, out_hbm,
               idx_v, nf_v, qf_v, obuf, sem):
    nqw = Nq // (NW // B)
    nblk = nqw // NQ_BLK
    cid = lax.axis_index("c")
    sid = lax.axis_index("s")
    wid = cid * NS + sid
    b = wid // 8
    q0 = (wid % 8) * nqw
    iota = lax.iota(jnp.int32, LANES)

    v = wid % 8

    def sub_block(s, c):
        nq = q0 + s * NQ_BLK
        pltpu.sync_copy(idx_hbm.at[b, v * nblk + s], idx_v)
        pltpu.async_copy(kf_hbm.at[idx_v], nf_v, sem).wait()
        pltpu.sync_copy(qf_hbm.at[b, pl.ds(nq, NQ_BLK), :], qf_v)

        def col(t, c1):
            nql = t // (C // LANES)
            c0 = (t % (C // LANES)) * LANES
            qv = qf_v[nql, pl.ds(c0, LANES)]
            rowc = c0 + iota
            nqlv = jnp.full((LANES,), nql, jnp.int32)
            for kk in range(KNN):
                v = nf_v[kk * NQ_BLK + nql, pl.ds(c0, LANES)]
                kv = jnp.full((LANES,), kk, jnp.int32)
                plsc.store_scatter(obuf, [rowc, nqlv, kv], v - qv)
                plsc.store_scatter(obuf, [rowc + C, nqlv, kv], qv)
            return c1
        lax.fori_loop(0, NQ_BLK * (C // LANES), col, 0)
        pltpu.sync_copy(obuf, out_hbm.at[b, :, pl.ds(nq, NQ_BLK), :])
        return c
    lax.fori_loop(0, nblk, sub_block, 0)


def _edge_sc(kf_t, qf_t, idx):
    B, Nq, C = qf_t.shape
    mesh = plsc.VectorSubcoreMesh(core_axis_name="c", subcore_axis_name="s")
    return pl.kernel(
        functools.partial(_edge_body, B, C, Nq),
        mesh=mesh,
        compiler_params=pltpu.CompilerParams(use_tc_tiling_on_sc=False, needs_layout_passes=False),
        out_type=jax.ShapeDtypeStruct((B, 2 * C, Nq, KNN), jnp.float32),
        scratch_types=[
            pltpu.VMEM((NQ_BLK * KNN,), jnp.int32),
            pltpu.VMEM((NQ_BLK * KNN, C), jnp.float32),
            pltpu.VMEM((NQ_BLK, C), jnp.float32),
            pltpu.VMEM((2 * C, NQ_BLK, KNN), jnp.float32),
            pltpu.SemaphoreType.DMA,
        ],
    )(kf_t, qf_t, idx)


def kernel(query_coords, query_features, key_coords, key_features):
    kf_t = _transpose_key(key_features)
    qf_t = _transpose_qf(query_features)
    idx = _knn_sc(query_coords, key_coords)
    return _edge_sc(kf_t, qf_t, idx)


# speedup vs baseline: 3.6070x; 1.4250x over previous
import functools

import jax
import jax.numpy as jnp
from jax import lax
from jax.experimental import pallas as pl
from jax.experimental.pallas import tpu as pltpu
from jax.experimental.pallas import tpu_sc as plsc

KNN = 16
LANES = 16
NC, NS = 2, 16
NW = NC * NS
QB = 8
NQ_BLK = 4

_GDN = lax.GatherDimensionNumbers(
    offset_dims=(), collapsed_slice_dims=(0,), start_index_map=(0,))


def _permute(v, idx):
    return lax.gather(v, idx[:, None], _GDN, (1,),
                      mode=lax.GatherScatterMode.PROMISE_IN_BOUNDS)


def _splat(v, lane):
    return _permute(v, jnp.full((LANES,), lane, jnp.int32))


def _tr_body(x_ref, o_ref):
    o_ref[...] = x_ref[0].T


def _transpose_key(key_features):
    B, C, Nk = key_features.shape
    TK = 512
    return pl.pallas_call(
        _tr_body,
        grid=(B, Nk // TK),
        in_specs=[pl.BlockSpec((1, C, TK), lambda b, t: (b, 0, t))],
        out_specs=pl.BlockSpec((TK, C), lambda b, t: (b * (Nk // TK) + t, 0)),
        out_shape=jax.ShapeDtypeStruct((B * Nk, C), jnp.float32),
    )(key_features)


def _tr_q_body(x_ref, o_ref):
    o_ref[0] = x_ref[0].T


def _transpose_qf(query_features):
    B, C, Nq = query_features.shape
    TQ = 512
    return pl.pallas_call(
        _tr_q_body,
        grid=(B, Nq // TQ),
        in_specs=[pl.BlockSpec((1, C, TQ), lambda b, t: (b, 0, t))],
        out_specs=pl.BlockSpec((1, TQ, C), lambda b, t: (b, t, 0)),
        out_shape=jax.ShapeDtypeStruct((B, Nq, C), jnp.float32),
    )(query_features)


def _knn_body(Nq, Nk, qc_hbm, kc_hbm, qcb_hbm, kcb_hbm, idx_hbm,
              kc_v, qc_v, kcb_v, qcb_v, kn2_v, pd_v, pi_v, oidx_v):
    nqw = Nq // (NW // 4)
    nchunk = Nk // LANES
    cid = lax.axis_index("c")
    sid = lax.axis_index("s")
    wid = cid * NS + sid
    b = wid // 8
    q0 = (wid % 8) * nqw
    pltpu.sync_copy(kc_hbm.at[b], kc_v)
    pltpu.sync_copy(qc_hbm.at[b, :, pl.ds(q0, nqw)], qc_v)
    pltpu.sync_copy(kcb_hbm.at[b], kcb_v)
    pltpu.sync_copy(qcb_hbm.at[b, :, pl.ds(q0, nqw)], qcb_v)
    iota = lax.iota(jnp.int32, LANES)
    iota_m1 = jnp.maximum(iota - 1, 0)
    lane15 = jnp.full((LANES,), 15, jnp.int32)
    inf = jnp.full((LANES,), jnp.inf, jnp.float32)
    zeros_i = jnp.zeros((LANES,), jnp.int32)

    def kn2_step(j, c):
        kx = kc_v[0, pl.ds(j * LANES, LANES)]
        ky = kc_v[1, pl.ds(j * LANES, LANES)]
        kz = kc_v[2, pl.ds(j * LANES, LANES)]
        kn2_v[pl.ds(j * LANES, LANES)] = (kx * kx + ky * ky) + kz * kz
        return c
    lax.fori_loop(0, nchunk, kn2_step, 0)

    def qgroup(qg, c0):
        base = (qg // 2) * LANES
        off = (qg % 2) * QB
        qxv = qc_v[0, pl.ds(base, LANES)]
        qyv = qc_v[1, pl.ds(base, LANES)]
        qzv = qc_v[2, pl.ds(base, LANES)]
        qxbv = qcb_v[0, pl.ds(base, LANES)]
        qybv = qcb_v[1, pl.ds(base, LANES)]
        qzbv = qcb_v[2, pl.ds(base, LANES)]
        qxb, qyb, qzb, qn2 = [], [], [], []
        for i in range(QB):
            qxb.append(_splat(qxbv, off + i))
            qyb.append(_splat(qybv, off + i))
            qzb.append(_splat(qzbv, off + i))
            qx = _splat(qxv, off + i)
            qy = _splat(qyv, off + i)
            qz = _splat(qzv, off + i)
            qn2.append((qx * qx + qy * qy) + qz * qz)
            pd_v[i] = inf
            pi_v[i] = zeros_i

        def dist(i, kx, ky, kz, kn2):
            qk = (kx * qxb[i] + ky * qyb[i]) + kz * qzb[i]
            return (qn2[i] - 2.0 * qk) + kn2

        def chunk_step(j, thrs):
            kx = kcb_v[0, pl.ds(j * LANES, LANES)]
            ky = kcb_v[1, pl.ds(j * LANES, LANES)]
            kz = kcb_v[2, pl.ds(j * LANES, LANES)]
            kn2 = kn2_v[pl.ds(j * LANES, LANES)]
            mor = dist(0, kx, ky, kz, kn2) < thrs[0]
            for i in range(1, QB):
                mor = mor | (dist(i, kx, ky, kz, kn2) < thrs[i])
            hit = jnp.any(mor)

            def dohit(thrs):
                ivec = iota + j * LANES
                out = []
                for i in range(QB):
                    d0 = dist(i, kx, ky, kz, kn2)

                    def wcond(carry):
                        d, thr = carry
                        return jnp.any(d < thr)

                    def wbody(carry, i=i):
                        d, thr = carry
                        lanev = plsc.all_reduce_ffs(d < thr)
                        cd = _permute(d, lanev)
                        ci = _permute(ivec, lanev)
                        pool_d = pd_v[i]
                        pool_i = pi_v[i]
                        mm = pool_d > cd
                        nd = jnp.where(mm, _permute(pool_d, iota_m1), pool_d)
                        ni = jnp.where(mm, _permute(pool_i, iota_m1), pool_i)
                        posv = plsc.all_reduce_ffs(mm)
                        pm = iota == posv
                        nd = jnp.where(pm, cd, nd)
                        ni = jnp.where(pm, ci, ni)
                        pd_v[i] = nd
                        pi_v[i] = ni
                        d = jnp.where(iota == lanev, jnp.inf, d)
                        return d, _splat(nd, 15)

                    _, nthr = lax.while_loop(wcond, wbody, (d0, thrs[i]))
                    out.append(nthr)
                return tuple(out)

            return lax.cond(hit, dohit, lambda t: t, thrs)

        lax.fori_loop(0, nchunk, chunk_step, (inf,) * QB)

        for i in range(QB):
            ql = qg * QB + i
            row = (ql * NQ_BLK) % (nqw * 2) + iota // NQ_BLK
            colv = (ql // 128) * NQ_BLK + iota % NQ_BLK
            plsc.store_scatter(oidx_v, [row, colv], pi_v[i] + b * Nk)
        return c0
    lax.fori_loop(0, nqw // QB, qgroup, 0)
    w = wid % 8
    pltpu.sync_copy(oidx_v, idx_hbm.at[b, :, pl.ds(w * QB, QB)])


def _knn_sc(query_coords, key_coords):
    B, _, Nq = query_coords.shape
    Nk = key_coords.shape[2]
    nqw = Nq // (NW // B)
    qcb = query_coords.astype(jnp.bfloat16).astype(jnp.float32)
    kcb = key_coords.astype(jnp.bfloat16).astype(jnp.float32)
    mesh = plsc.VectorSubcoreMesh(core_axis_name="c", subcore_axis_name="s")
    return pl.kernel(
        functools.partial(_knn_body, Nq, Nk),
        mesh=mesh,
        compiler_params=pltpu.CompilerParams(
            use_tc_tiling_on_sc=False, needs_layout_passes=False),
        out_type=jax.ShapeDtypeStruct((B, nqw * 2, 64), jnp.int32),
        scratch_types=[
            pltpu.VMEM((3, Nk), jnp.float32),
            pltpu.VMEM((3, nqw), jnp.float32),
            pltpu.VMEM((3, Nk), jnp.float32),
            pltpu.VMEM((3, nqw), jnp.float32),
            pltpu.VMEM((Nk,), jnp.float32),
            pltpu.VMEM((QB, LANES), jnp.float32),
            pltpu.VMEM((QB, LANES), jnp.int32),
            pltpu.VMEM((nqw * 2, QB), jnp.int32),
        ],
    )(query_coords, key_coords, qcb, kcb)


def _edge_body(B, C, Nq, kf_hbm, qf_hbm, idx_hbm, out_hbm,
               idx_v0, idx_v1, nf_v0, nf_v1, qf_v, obuf0, obuf1,
               gsem0, gsem1, osem0, osem1):
    nqw = Nq // (NW // B)
    nblk = nqw // NQ_BLK
    cid = lax.axis_index("c")
    sid = lax.axis_index("s")
    wid = cid * NS + sid
    b = wid // 8
    q0 = (wid % 8) * nqw
    v = wid % 8
    iota = lax.iota(jnp.int32, LANES)
    idx_vs = (idx_v0, idx_v1)
    nf_vs = (nf_v0, nf_v1)
    obufs = (obuf0, obuf1)
    gsems = (gsem0, gsem1)
    osems = (osem0, osem1)

    pltpu.sync_copy(idx_hbm.at[b, v * nblk], idx_v0)
    pltpu.async_copy(kf_hbm.at[idx_v0], nf_v0, gsem0)

    def pair(sp, c):
        for sl in range(2):
            s = sp * 2 + sl
            nq = q0 + s * NQ_BLK
            @pl.when(s + 1 < nblk)
            def _():
                pltpu.sync_copy(idx_hbm.at[b, v * nblk + s + 1], idx_vs[1 - sl])
                pltpu.async_copy(kf_hbm.at[idx_vs[1 - sl]], nf_vs[1 - sl],
                                 gsems[1 - sl])
            pltpu.sync_copy(qf_hbm.at[b, pl.ds(nq, NQ_BLK), :], qf_v)
            pltpu.make_async_copy(kf_hbm.at[idx_vs[sl]], nf_vs[sl],
                                  gsems[sl]).wait()
            @pl.when(s >= 2)
            def _():
                pltpu.make_async_copy(
                    obufs[sl], out_hbm.at[b, :, pl.ds(nq, NQ_BLK), :],
                    osems[sl]).wait()

            nf = nf_vs[sl]
            ob = obufs[sl]

            def col(t, c1):
                nql = t // (C // LANES)
                c0 = (t % (C // LANES)) * LANES
                qv = qf_v[nql, pl.ds(c0, LANES)]
                rowc = c0 + iota
                nqlv = jnp.full((LANES,), nql, jnp.int32)
                for kk in range(KNN):
                    vv = nf[kk * NQ_BLK + nql, pl.ds(c0, LANES)]
                    kv = jnp.full((LANES,), kk, jnp.int32)
                    plsc.store_scatter(ob, [rowc, nqlv, kv], vv - qv)
                    plsc.store_scatter(ob, [rowc + C, nqlv, kv], qv)
                return c1
            lax.fori_loop(0, NQ_BLK * (C // LANES), col, 0)
            pltpu.async_copy(ob, out_hbm.at[b, :, pl.ds(nq, NQ_BLK), :],
                             osems[sl])
        return c
    lax.fori_loop(0, nblk // 2, pair, 0)
    for sl in range(2):
        s = nblk - 2 + sl
        nq = q0 + s * NQ_BLK
        pltpu.make_async_copy(
            obufs[sl], out_hbm.at[b, :, pl.ds(nq, NQ_BLK), :],
            osems[sl]).wait()


def _edge_sc(kf_t, qf_t, idx):
    B, Nq, C = qf_t.shape
    mesh = plsc.VectorSubcoreMesh(core_axis_name="c", subcore_axis_name="s")
    return pl.kernel(
        functools.partial(_edge_body, B, C, Nq),
        mesh=mesh,
        compiler_params=pltpu.CompilerParams(
            use_tc_tiling_on_sc=False, needs_layout_passes=False),
        out_type=jax.ShapeDtypeStruct((B, 2 * C, Nq, KNN), jnp.float32),
        scratch_types=[
            pltpu.VMEM((NQ_BLK * KNN,), jnp.int32),
            pltpu.VMEM((NQ_BLK * KNN,), jnp.int32),
            pltpu.VMEM((NQ_BLK * KNN, C), jnp.float32),
            pltpu.VMEM((NQ_BLK * KNN, C), jnp.float32),
            pltpu.VMEM((NQ_BLK, C), jnp.float32),
            pltpu.VMEM((2 * C, NQ_BLK, KNN), jnp.float32),
            pltpu.VMEM((2 * C, NQ_BLK, KNN), jnp.float32),
            pltpu.SemaphoreType.DMA,
            pltpu.SemaphoreType.DMA,
            pltpu.SemaphoreType.DMA,
            pltpu.SemaphoreType.DMA,
        ],
    )(kf_t, qf_t, idx)


def kernel(query_coords, query_features, key_coords, key_features):
    kf_t = _transpose_key(key_features)
    qf_t = _transpose_qf(query_features)
    idx = _knn_sc(query_coords, key_coords)
    return _edge_sc(kf_t, qf_t, idx)
